# baseline (speedup 1.0000x reference)
import jax
import jax.numpy as jnp
from jax.experimental import pallas as pl
from jax.experimental.pallas import tpu as pltpu

_CH = 128
_NBUF = 8


def _router_body(x_hbm, w_ref, b_ref, o_ref, xbuf, insem):
    n = x_hbm.shape[0]
    nchunks = n // _CH

    def in_copy(i, slot):
        return pltpu.make_async_copy(
            x_hbm.at[pl.ds(i * _CH, _CH), :], xbuf.at[slot], insem.at[slot]
        )

    for j in range(_NBUF - 1):
        in_copy(j, j).start()

    def step(i, carry):
        slot = jax.lax.rem(i, _NBUF)
        ahead = i + _NBUF - 1

        @pl.when(ahead < nchunks)
        def _():
            in_copy(ahead, jax.lax.rem(ahead, _NBUF)).start()

        in_copy(i, slot).wait()
        logits = jnp.dot(
            xbuf[slot], w_ref[...], preferred_element_type=jnp.float32
        )
        logits = logits + b_ref[...].reshape(1, -1)
        m = jnp.max(logits, axis=-1, keepdims=True)
        e = jnp.exp(logits - m)
        o_ref[pl.ds(i * _CH, _CH), :] = e * (
            1.0 / jnp.sum(e, axis=-1, keepdims=True)
        )
        return carry

    jax.lax.fori_loop(0, nchunks, step, 0, unroll=False)


def kernel(x, W, b):
    n, k = x.shape
    ne = W.shape[1]
    return pl.pallas_call(
        _router_body,
        in_specs=[
            pl.BlockSpec(memory_space=pltpu.MemorySpace.HBM),
            pl.BlockSpec(memory_space=pltpu.MemorySpace.VMEM),
            pl.BlockSpec(memory_space=pltpu.MemorySpace.VMEM),
        ],
        out_specs=pl.BlockSpec(memory_space=pltpu.MemorySpace.VMEM),
        out_shape=jax.ShapeDtypeStruct((n, ne), jnp.float32),
        scratch_shapes=[
            pltpu.VMEM((_NBUF, _CH, k), jnp.float32),
            pltpu.SemaphoreType.DMA((_NBUF,)),
        ],
    )(x, W, b)


# speedup vs baseline: 1.0668x; 1.0668x over previous
import jax
import jax.numpy as jnp
from jax.experimental import pallas as pl
from jax.experimental.pallas import tpu as pltpu

_CH = 256
_NBUF = 4


def _router_body(x_hbm, w_ref, b_ref, o_ref, xbuf, insem):
    n = x_hbm.shape[0]
    nchunks = n // _CH

    def in_copy(i, slot):
        return pltpu.make_async_copy(
            x_hbm.at[pl.ds(i * _CH, _CH), :], xbuf.at[slot], insem.at[slot]
        )

    for j in range(_NBUF):
        in_copy(j, j).start()

    def step(i, carry):
        slot = jax.lax.rem(i, _NBUF)
        in_copy(i, slot).wait()
        logits = jnp.dot(
            xbuf[slot], w_ref[...], preferred_element_type=jnp.float32
        )
        logits = logits + b_ref[...].reshape(1, -1)
        m = jnp.max(logits, axis=-1, keepdims=True)
        e = jnp.exp(logits - m)
        o_ref[pl.ds(i * _CH, _CH), :] = e * (
            1.0 / jnp.sum(e, axis=-1, keepdims=True)
        )

        @pl.when(i + _NBUF < nchunks)
        def _():
            in_copy(i + _NBUF, slot).start()

        return carry

    jax.lax.fori_loop(0, nchunks, step, 0, unroll=False)


def kernel(x, W, b):
    n, k = x.shape
    ne = W.shape[1]
    return pl.pallas_call(
        _router_body,
        in_specs=[
            pl.BlockSpec(memory_space=pltpu.MemorySpace.HBM),
            pl.BlockSpec(memory_space=pltpu.MemorySpace.VMEM),
            pl.BlockSpec(memory_space=pltpu.MemorySpace.VMEM),
        ],
        out_specs=pl.BlockSpec(memory_space=pltpu.MemorySpace.VMEM),
        out_shape=jax.ShapeDtypeStruct((n, ne), jnp.float32),
        scratch_shapes=[
            pltpu.VMEM((_NBUF, _CH, k), jnp.float32),
            pltpu.SemaphoreType.DMA((_NBUF,)),
        ],
    )(x, W, b)
